# Initial kernel scaffold; baseline (speedup 1.0000x reference)
import functools

import jax
import jax.numpy as jnp
from jax import lax
from jax.experimental import pallas as pl
from jax.experimental.pallas import tpu as pltpu
from jax.experimental.pallas import tpu_sc as plsc


@functools.lru_cache(maxsize=None)
def _build(n_total: int, vocab: int, dim: int):
  info = plsc.get_sparse_core_info()
  num_workers = info.num_cores * info.num_subcores
  assert n_total % num_workers == 0
  n_per_w = n_total // num_workers
  chunk = 1600
  assert n_per_w % chunk == 0
  n_chunks = n_per_w // chunk

  mesh = plsc.VectorSubcoreMesh(core_axis_name="c", subcore_axis_name="s")

  @functools.partial(
      pl.kernel,
      mesh=mesh,
      out_type=jax.ShapeDtypeStruct((n_total, dim), jnp.float32),
      scratch_types=[
          pltpu.VMEM((chunk,), jnp.int32),
          pltpu.VMEM((chunk, dim), jnp.float32),
          pltpu.SemaphoreType.DMA,
      ],
  )
  def gather_kernel(idx_hbm, table_hbm, out_hbm, idx_v, rows_v, sem):
    wid = lax.axis_index("s") * info.num_cores + lax.axis_index("c")
    base = wid * n_per_w

    def body(i, carry):
      off = base + i * chunk
      pltpu.sync_copy(idx_hbm.at[pl.ds(off, chunk)], idx_v)
      pltpu.async_copy(table_hbm.at[idx_v], rows_v, sem).wait()
      pltpu.sync_copy(rows_v, out_hbm.at[pl.ds(off, chunk)])
      return carry

    lax.fori_loop(0, n_chunks, body, 0)

  return gather_kernel


def kernel(input, weight):
  batch, hist = input.shape
  vocab, dim = weight.shape
  idx = input.reshape(-1).astype(jnp.int32)
  out = _build(batch * hist, vocab, dim)(idx, weight)
  return out.reshape(batch, hist, dim)


# speedup vs baseline: 1.1030x; 1.1030x over previous
import functools

import jax
import jax.numpy as jnp
from jax import lax
from jax.experimental import pallas as pl
from jax.experimental.pallas import tpu as pltpu
from jax.experimental.pallas import tpu_sc as plsc


@functools.lru_cache(maxsize=None)
def _build(n_total: int, vocab: int, dim: int):
  info = plsc.get_sparse_core_info()
  num_workers = info.num_cores * info.num_subcores
  assert n_total % num_workers == 0
  n_per_w = n_total // num_workers
  chunk = 1600
  assert n_per_w % chunk == 0
  n_chunks = n_per_w // chunk

  mesh = plsc.VectorSubcoreMesh(core_axis_name="c", subcore_axis_name="s")

  @functools.partial(
      pl.kernel,
      mesh=mesh,
      out_type=jax.ShapeDtypeStruct((n_total, dim), jnp.float32),
      scratch_types=[
          pltpu.VMEM((chunk,), jnp.int32),
          pltpu.VMEM((chunk, dim), jnp.float32),
          pltpu.SemaphoreType.DMA,
      ],
      compiler_params=pltpu.CompilerParams(use_tc_tiling_on_sc=False),
  )
  def gather_kernel(idx_hbm, table_hbm, out_hbm, idx_v, rows_v, sem):
    wid = lax.axis_index("s") * info.num_cores + lax.axis_index("c")
    base = wid * n_per_w

    def body(i, carry):
      off = base + i * chunk
      pltpu.sync_copy(idx_hbm.at[pl.ds(off, chunk)], idx_v)
      pltpu.async_copy(table_hbm.at[idx_v], rows_v, sem).wait()
      pltpu.sync_copy(rows_v, out_hbm.at[pl.ds(off, chunk)])
      return carry

    lax.fori_loop(0, n_chunks, body, 0)

  return gather_kernel


def kernel(input, weight):
  batch, hist = input.shape
  vocab, dim = weight.shape
  idx = input.reshape(-1).astype(jnp.int32)
  out = _build(batch * hist, vocab, dim)(idx, weight)
  return out.reshape(batch, hist, dim)


# speedup vs baseline: 1.1101x; 1.0064x over previous
import functools

import jax
import jax.numpy as jnp
from jax import lax
from jax.experimental import pallas as pl
from jax.experimental.pallas import tpu as pltpu
from jax.experimental.pallas import tpu_sc as plsc

_NBUF = 2


@functools.lru_cache(maxsize=None)
def _build(n_total: int, vocab: int, dim: int):
  info = plsc.get_sparse_core_info()
  num_workers = info.num_cores * info.num_subcores
  assert n_total % num_workers == 0
  n_per_w = n_total // num_workers
  chunk = 1280
  assert n_per_w % (chunk * _NBUF) == 0
  n_chunks = n_per_w // chunk
  n_outer = n_chunks // _NBUF

  mesh = plsc.VectorSubcoreMesh(core_axis_name="c", subcore_axis_name="s")

  @functools.partial(
      pl.kernel,
      mesh=mesh,
      out_type=jax.ShapeDtypeStruct((n_total, dim), jnp.float32),
      scratch_types=[
          pltpu.VMEM((n_per_w,), jnp.int32),
          *[pltpu.VMEM((chunk, dim), jnp.float32) for _ in range(_NBUF)],
          pltpu.SemaphoreType.DMA,
          *[pltpu.SemaphoreType.DMA for _ in range(2 * _NBUF)],
      ],
      compiler_params=pltpu.CompilerParams(use_tc_tiling_on_sc=False),
  )
  def gather_kernel(idx_hbm, table_hbm, out_hbm, idx_v, *bufs_and_sems):
    rows = bufs_and_sems[:_NBUF]
    sem_i = bufs_and_sems[_NBUF]
    sg = bufs_and_sems[_NBUF + 1:2 * _NBUF + 1]
    sw = bufs_and_sems[2 * _NBUF + 1:]

    wid = lax.axis_index("s") * info.num_cores + lax.axis_index("c")
    base = wid * n_per_w

    pltpu.async_copy(idx_hbm.at[pl.ds(base, n_per_w)], idx_v, sem_i).wait()

    def gather_chunk(i, b):
      pltpu.async_copy(
          table_hbm.at[idx_v.at[pl.ds(i * chunk, chunk)]], rows[b], sg[b])

    def writeback_chunk(i, b):
      pltpu.async_copy(rows[b], out_hbm.at[pl.ds(base + i * chunk, chunk)],
                       sw[b])

    for b in range(_NBUF):
      gather_chunk(b, b)

    def outer(g, carry):
      i0 = g * _NBUF
      for b in range(_NBUF):
        i = i0 + b
        pltpu.make_async_copy(
            table_hbm.at[idx_v.at[pl.ds(i * chunk, chunk)]], rows[b],
            sg[b]).wait()
        writeback_chunk(i, b)
      for b in range(_NBUF):
        i = i0 + b + _NBUF

        @pl.when(i < n_chunks)
        def _():
          pltpu.make_async_copy(
              rows[b], out_hbm.at[pl.ds(base + (i - _NBUF) * chunk, chunk)],
              sw[b]).wait()
          gather_chunk(i, b)

      return carry

    lax.fori_loop(0, n_outer, outer, 0)

    for b in range(_NBUF):
      i = n_chunks - _NBUF + b
      pltpu.make_async_copy(
          rows[b], out_hbm.at[pl.ds(base + i * chunk, chunk)], sw[b]).wait()

  return gather_kernel


def kernel(input, weight):
  batch, hist = input.shape
  vocab, dim = weight.shape
  idx = input.reshape(-1).astype(jnp.int32)
  out = _build(batch * hist, vocab, dim)(idx, weight)
  return out.reshape(batch, hist, dim)


# speedup vs baseline: 1.5338x; 1.3817x over previous
import functools

import jax
import jax.numpy as jnp
from jax import lax
from jax.experimental import pallas as pl
from jax.experimental.pallas import tpu as pltpu
from jax.experimental.pallas import tpu_sc as plsc

_LANES = 16


@functools.lru_cache(maxsize=None)
def _build(batch: int, hist: int, vocab: int, dim: int):
  info = plsc.get_sparse_core_info()
  num_workers = info.num_cores * info.num_subcores
  assert batch % num_workers == 0
  bpt = batch // num_workers
  assert bpt % _LANES == 0 and dim == 2 * _LANES

  mesh = plsc.VectorSubcoreMesh(core_axis_name="c", subcore_axis_name="s")

  @functools.partial(
      pl.kernel,
      mesh=mesh,
      out_type=jax.ShapeDtypeStruct((hist, dim, batch), jnp.float32),
      scratch_types=[
          pltpu.VMEM((bpt * hist,), jnp.int32),
          pltpu.VMEM((bpt,), jnp.int32),
          pltpu.VMEM((bpt, dim), jnp.float32),
          pltpu.VMEM((dim, bpt), jnp.float32),
          pltpu.SemaphoreType.DMA,
          pltpu.SemaphoreType.DMA,
      ],
      compiler_params=pltpu.CompilerParams(
          use_tc_tiling_on_sc=False, needs_layout_passes=False),
  )
  def gather_kernel(idx_hbm, table_hbm, out_hbm, idx_v, idx_col, rows_v,
                    out_t, sem_g, sem_w):
    wid = lax.axis_index("s") * info.num_cores + lax.axis_index("c")
    b0 = wid * bpt
    iota = lax.iota(jnp.int32, _LANES)

    pltpu.sync_copy(idx_hbm.at[pl.ds(b0 * hist, bpt * hist)], idx_v)

    def per_h(h, carry):
      for g in range(bpt // _LANES):
        vals = plsc.load_gather(
            idx_v, [(iota + (g * _LANES)) * hist + h])
        idx_col[pl.ds(g * _LANES, _LANES)] = vals

      pltpu.async_copy(table_hbm.at[idx_col], rows_v, sem_g).wait()

      def trans(j4, carry2):
        for u in range(4):
          j = j4 * 4 + u
          v_lo = rows_v[j, pl.ds(0, _LANES)]
          v_hi = rows_v[j, pl.ds(_LANES, _LANES)]
          jvec = jnp.full((_LANES,), 0, jnp.int32) + j
          plsc.store_scatter(out_t, [iota, jvec], v_lo)
          plsc.store_scatter(out_t, [iota + _LANES, jvec], v_hi)
        return carry2

      lax.fori_loop(0, bpt // 4, trans, 0)

      pltpu.async_copy(out_t, out_hbm.at[h, :, pl.ds(b0, bpt)], sem_w).wait()
      return carry

    lax.fori_loop(0, hist, per_h, 0)

  return gather_kernel


def kernel(input, weight):
  batch, hist = input.shape
  vocab, dim = weight.shape
  idx = input.reshape(-1).astype(jnp.int32)
  out_t = _build(batch, hist, vocab, dim)(idx, weight)
  return jnp.transpose(out_t, (2, 0, 1))


# speedup vs baseline: 1.6672x; 1.0870x over previous
import functools

import jax
import jax.numpy as jnp
from jax import lax
from jax.experimental import pallas as pl
from jax.experimental.pallas import tpu as pltpu
from jax.experimental.pallas import tpu_sc as plsc

_LANES = 16


@functools.lru_cache(maxsize=None)
def _build(batch: int, hist: int, vocab: int, dim: int):
  info = plsc.get_sparse_core_info()
  num_workers = info.num_cores * info.num_subcores
  assert batch % num_workers == 0
  bpt = batch // num_workers
  assert bpt % _LANES == 0 and dim == 2 * _LANES

  mesh = plsc.VectorSubcoreMesh(core_axis_name="c", subcore_axis_name="s")

  @functools.partial(
      pl.kernel,
      mesh=mesh,
      out_type=jax.ShapeDtypeStruct((hist, dim, batch), jnp.float32),
      scratch_types=[
          pltpu.VMEM((bpt * hist,), jnp.int32),
          *[pltpu.VMEM((bpt,), jnp.int32) for _ in range(2)],
          *[pltpu.VMEM((bpt, dim), jnp.float32) for _ in range(2)],
          *[pltpu.VMEM((dim, bpt), jnp.float32) for _ in range(2)],
          *[pltpu.SemaphoreType.DMA for _ in range(4)],
      ],
      compiler_params=pltpu.CompilerParams(
          use_tc_tiling_on_sc=False, needs_layout_passes=False),
  )
  def gather_kernel(idx_hbm, table_hbm, out_hbm, idx_v, ic0, ic1, rv0, rv1,
                    ot0, ot1, sg0, sg1, sw0, sw1):
    idx_col = (ic0, ic1)
    rows_v = (rv0, rv1)
    out_t = (ot0, ot1)
    sem_g = (sg0, sg1)
    sem_w = (sw0, sw1)
    wid = lax.axis_index("s") * info.num_cores + lax.axis_index("c")
    b0 = wid * bpt
    iota = lax.iota(jnp.int32, _LANES)

    pltpu.sync_copy(idx_hbm.at[pl.ds(b0 * hist, bpt * hist)], idx_v)

    def build_idx_col(h, p):
      for g in range(bpt // _LANES):
        vals = plsc.load_gather(idx_v, [(iota + (g * _LANES)) * hist + h])
        idx_col[p][pl.ds(g * _LANES, _LANES)] = vals

    def start_gather(p):
      pltpu.async_copy(table_hbm.at[idx_col[p]], rows_v[p], sem_g[p])

    for p in range(2):
      build_idx_col(p, p)
      start_gather(p)

    def per_pair(g, carry):
      for p in range(2):
        h = g * 2 + p
        pltpu.make_async_copy(table_hbm.at[idx_col[p]], rows_v[p],
                              sem_g[p]).wait()

        @pl.when(g >= 1)
        def _():
          pltpu.make_async_copy(out_t[p],
                                out_hbm.at[h - 2, :, pl.ds(b0, bpt)],
                                sem_w[p]).wait()

        def trans(j4, carry2):
          for u in range(4):
            j = j4 * 4 + u
            v_lo = rows_v[p][j, pl.ds(0, _LANES)]
            v_hi = rows_v[p][j, pl.ds(_LANES, _LANES)]
            jvec = jnp.full((_LANES,), 0, jnp.int32) + j
            plsc.store_scatter(out_t[p], [iota, jvec], v_lo)
            plsc.store_scatter(out_t[p], [iota + _LANES, jvec], v_hi)
          return carry2

        lax.fori_loop(0, bpt // 4, trans, 0)

        pltpu.async_copy(out_t[p], out_hbm.at[h, :, pl.ds(b0, bpt)],
                         sem_w[p])

        @pl.when(g < (hist // 2) - 1)
        def _():
          build_idx_col(h + 2, p)
          start_gather(p)

      return carry

    lax.fori_loop(0, hist // 2, per_pair, 0)

    for p in range(2):
      pltpu.make_async_copy(out_t[p], out_hbm.at[hist - 2 + p, :,
                                                 pl.ds(b0, bpt)],
                            sem_w[p]).wait()

  return gather_kernel


def kernel(input, weight):
  batch, hist = input.shape
  vocab, dim = weight.shape
  idx = input.reshape(-1).astype(jnp.int32)
  out_t = _build(batch, hist, vocab, dim)(idx, weight)
  return jnp.transpose(out_t, (2, 0, 1))
